# baseline (device time: 47991 ns/iter reference)
import jax
import jax.numpy as jnp
from jax import lax
from jax.experimental import pallas as pl
from jax.experimental.pallas import tpu as pltpu

N_DEV = 4
F8 = jnp.float8_e4m3fn

ABLATE_NO_COMM = False

SEND_ORDER = (2, 1, 3)
COMPUTE_ORDER = (0, 1, 3, 2)


def kernel(x, w_mat, scale_x, scale_w):
    m_total, k_shard = x.shape
    k_total, n = w_mat.shape
    m_per = m_total // N_DEV
    k_per = k_total // N_DEV

    def body(x_hbm_ref, w_hbm_ref, sx_ref, sw_ref, out_hbm_ref,
             xst_ref, xs_ref, xg_ref, wst_ref, w8_ref, acc_ref,
             x_sems, w_sems, out_sem, send_sems, recv_sems):
        my = lax.axis_index("i")

        peers = [(my + d) % N_DEV for d in SEND_ORDER]
        x_copies = []
        for t, p in enumerate(peers + [my]):
            cp = pltpu.make_async_copy(
                x_hbm_ref.at[pl.ds(p * m_per, m_per), :],
                xst_ref.at[t],
                x_sems.at[t],
            )
            cp.start()
            x_copies.append(cp)

        blocks = [(my + d) % N_DEV for d in COMPUTE_ORDER]
        w_copies = [
            pltpu.make_async_copy(
                w_hbm_ref.at[pl.ds(o * k_per, k_per), :],
                wst_ref.at[t % 2],
                w_sems.at[t % 2],
            )
            for t, o in enumerate(blocks)
        ]
        w_copies[0].start()
        w_copies[1].start()

        barrier_sem = pltpu.get_barrier_semaphore()
        for j in range(N_DEV):
            @pl.when(j != my)
            def _():
                pl.semaphore_signal(
                    barrier_sem, inc=1,
                    device_id=(j,), device_id_type=pl.DeviceIdType.MESH,
                )
        pl.semaphore_wait(barrier_sem, N_DEV - 1)

        sends = []
        for t, p in enumerate(peers):
            x_copies[t].wait()
            xs_ref[t] = xst_ref[t].astype(F8)
            rdma = pltpu.make_async_remote_copy(
                src_ref=xs_ref.at[t],
                dst_ref=xg_ref.at[my],
                send_sem=send_sems.at[t],
                recv_sem=recv_sems.at[my],
                device_id=(p,),
                device_id_type=pl.DeviceIdType.MESH,
            )
            if not ABLATE_NO_COMM:
                rdma.start()
                sends.append(rdma)

        x_copies[3].wait()
        xg_ref[my] = xst_ref[3].astype(F8)

        scale = sx_ref[0] * sw_ref[0]
        for t, o in enumerate(blocks):
            w_copies[t].wait()
            w8_ref[t] = wst_ref[t % 2].astype(F8)
            if t + 2 < N_DEV:
                w_copies[t + 2].start()

            if t > 0 and not ABLATE_NO_COMM:
                recv = pltpu.make_async_remote_copy(
                    src_ref=xs_ref.at[0],
                    dst_ref=xg_ref.at[o],
                    send_sem=send_sems.at[3],
                    recv_sem=recv_sems.at[o],
                    device_id=(o,),
                    device_id_type=pl.DeviceIdType.MESH,
                )
                recv.wait_recv()

            contrib = lax.dot_general(
                xg_ref[my if ABLATE_NO_COMM else o], w8_ref[t],
                dimension_numbers=(((1,), (0,)), ((), ())),
                preferred_element_type=jnp.float32,
            ) * scale
            if t == 0:
                acc_ref[:, :] = contrib
            else:
                acc_ref[:, :] = acc_ref[:, :] + contrib

        out_cp = pltpu.make_async_copy(acc_ref, out_hbm_ref, out_sem)
        out_cp.start()
        for rdma in sends:
            rdma.wait_send()
        out_cp.wait()

    return pl.pallas_call(
        body,
        out_shape=jax.ShapeDtypeStruct((m_per, n), jnp.float32),
        in_specs=[
            pl.BlockSpec(memory_space=pl.ANY),
            pl.BlockSpec(memory_space=pl.ANY),
            pl.BlockSpec(memory_space=pltpu.SMEM),
            pl.BlockSpec(memory_space=pltpu.SMEM),
        ],
        out_specs=pl.BlockSpec(memory_space=pl.ANY),
        scratch_shapes=[
            pltpu.VMEM((N_DEV, m_per, k_shard), jnp.float32),
            pltpu.VMEM((N_DEV - 1, m_per, k_shard), F8),
            pltpu.VMEM((N_DEV, m_per, k_per), F8),
            pltpu.VMEM((2, k_per, n), jnp.float32),
            pltpu.VMEM((N_DEV, k_per, n), F8),
            pltpu.VMEM((m_per, n), jnp.float32),
            pltpu.SemaphoreType.DMA((N_DEV,)),
            pltpu.SemaphoreType.DMA((2,)),
            pltpu.SemaphoreType.DMA,
            pltpu.SemaphoreType.DMA((N_DEV,)),
            pltpu.SemaphoreType.DMA((N_DEV,)),
        ],
        compiler_params=pltpu.CompilerParams(
            collective_id=0,
            vmem_limit_bytes=64 * 1024 * 1024,
        ),
    )(x, w_mat, scale_x, scale_w)


# device time: 47048 ns/iter; 1.0200x vs baseline; 1.0200x over previous
import jax
import jax.numpy as jnp
from jax import lax
from jax.experimental import pallas as pl
from jax.experimental.pallas import tpu as pltpu

N_DEV = 4
F8 = jnp.float8_e4m3fn

ABLATE_NO_COMM = False

SEND_ORDER = (2, 1, 3)
COMPUTE_ORDER = (0, 1, 3, 2)


def kernel(x, w_mat, scale_x, scale_w):
    m_total, k_shard = x.shape
    k_total, n = w_mat.shape
    m_per = m_total // N_DEV
    k_per = k_total // N_DEV

    def body(x_hbm_ref, w_hbm_ref, sx_ref, sw_ref, out_hbm_ref,
             xst_ref, xs_ref, xg_ref, wst_ref, w8_ref, acc_ref,
             x_sems, w_sems, out_sems, send_sems, recv_sems, recv2_sems):
        my = lax.axis_index("i")

        peers = [(my + d) % N_DEV for d in SEND_ORDER]
        x_copies = []
        for t, p in enumerate(peers + [my]):
            cp = pltpu.make_async_copy(
                x_hbm_ref.at[pl.ds(p * m_per, m_per), :],
                xst_ref.at[t],
                x_sems.at[t],
            )
            cp.start()
            x_copies.append(cp)

        blocks = [(my + d) % N_DEV for d in COMPUTE_ORDER]
        w_copies = [
            pltpu.make_async_copy(
                w_hbm_ref.at[pl.ds(o * k_per, k_per), :],
                wst_ref.at[t % 2],
                w_sems.at[t % 2],
            )
            for t, o in enumerate(blocks)
        ]
        w_copies[0].start()
        w_copies[1].start()

        barrier_sem = pltpu.get_barrier_semaphore()
        for j in range(N_DEV):
            @pl.when(j != my)
            def _():
                pl.semaphore_signal(
                    barrier_sem, inc=1,
                    device_id=(j,), device_id_type=pl.DeviceIdType.MESH,
                )
        pl.semaphore_wait(barrier_sem, N_DEV - 1)

        half = m_per // 2
        sends = []
        for t, p in enumerate(peers):
            x_copies[t].wait()
            xs_ref[t] = xst_ref[t].astype(F8)
            if t == 0:
                for h_i, (h_lo, sem, rsem) in enumerate(
                    [(0, send_sems.at[0], recv_sems.at[my]),
                     (half, send_sems.at[1], recv2_sems.at[my])]
                ):
                    rdma = pltpu.make_async_remote_copy(
                        src_ref=xs_ref.at[0, pl.ds(h_lo, half)],
                        dst_ref=xg_ref.at[my, pl.ds(h_lo, half)],
                        send_sem=sem,
                        recv_sem=rsem,
                        device_id=(p,),
                        device_id_type=pl.DeviceIdType.MESH,
                    )
                    if not ABLATE_NO_COMM:
                        rdma.start()
                        sends.append(rdma)
            else:
                rdma = pltpu.make_async_remote_copy(
                    src_ref=xs_ref.at[t],
                    dst_ref=xg_ref.at[my],
                    send_sem=send_sems.at[t + 1],
                    recv_sem=recv_sems.at[my],
                    device_id=(p,),
                    device_id_type=pl.DeviceIdType.MESH,
                )
                if not ABLATE_NO_COMM:
                    rdma.start()
                    sends.append(rdma)

        x_copies[3].wait()
        xg_ref[my] = xst_ref[3].astype(F8)

        scale = sx_ref[0] * sw_ref[0]

        def block_dot(o, t, row_lo, rows):
            src = my if ABLATE_NO_COMM else o
            return lax.dot_general(
                xg_ref[src, pl.ds(row_lo, rows)], w8_ref[t],
                dimension_numbers=(((1,), (0,)), ((), ())),
                preferred_element_type=jnp.float32,
            ) * scale

        for t, o in enumerate(blocks[:3]):
            w_copies[t].wait()
            w8_ref[t] = wst_ref[t % 2].astype(F8)
            if t + 2 < N_DEV:
                w_copies[t + 2].start()

            if t > 0 and not ABLATE_NO_COMM:
                recv = pltpu.make_async_remote_copy(
                    src_ref=xs_ref.at[0],
                    dst_ref=xg_ref.at[o],
                    send_sem=send_sems.at[0],
                    recv_sem=recv_sems.at[o],
                    device_id=(o,),
                    device_id_type=pl.DeviceIdType.MESH,
                )
                recv.wait_recv()

            contrib = block_dot(o, t, 0, m_per)
            if t == 0:
                acc_ref[:, :] = contrib
            else:
                acc_ref[:, :] = acc_ref[:, :] + contrib

        o = blocks[3]
        w_copies[3].wait()
        w8_ref[3] = wst_ref[3 % 2].astype(F8)
        out_cps = [
            pltpu.make_async_copy(
                acc_ref.at[pl.ds(h_lo, half)],
                out_hbm_ref.at[pl.ds(h_lo, half)],
                out_sems.at[h_i],
            )
            for h_i, h_lo in enumerate((0, half))
        ]
        for h_i, (h_lo, rsems) in enumerate([(0, recv_sems), (half, recv2_sems)]):
            if not ABLATE_NO_COMM:
                recv = pltpu.make_async_remote_copy(
                    src_ref=xs_ref.at[0, pl.ds(h_lo, half)],
                    dst_ref=xg_ref.at[o, pl.ds(h_lo, half)],
                    send_sem=send_sems.at[0],
                    recv_sem=rsems.at[o],
                    device_id=(o,),
                    device_id_type=pl.DeviceIdType.MESH,
                )
                recv.wait_recv()
            acc_ref[pl.ds(h_lo, half), :] = (
                acc_ref[pl.ds(h_lo, half), :] + block_dot(o, 3, h_lo, half)
            )
            out_cps[h_i].start()

        for rdma in sends:
            rdma.wait_send()
        for cp in out_cps:
            cp.wait()

    return pl.pallas_call(
        body,
        out_shape=jax.ShapeDtypeStruct((m_per, n), jnp.float32),
        in_specs=[
            pl.BlockSpec(memory_space=pl.ANY),
            pl.BlockSpec(memory_space=pl.ANY),
            pl.BlockSpec(memory_space=pltpu.SMEM),
            pl.BlockSpec(memory_space=pltpu.SMEM),
        ],
        out_specs=pl.BlockSpec(memory_space=pl.ANY),
        scratch_shapes=[
            pltpu.VMEM((N_DEV, m_per, k_shard), jnp.float32),
            pltpu.VMEM((N_DEV - 1, m_per, k_shard), F8),
            pltpu.VMEM((N_DEV, m_per, k_per), F8),
            pltpu.VMEM((2, k_per, n), jnp.float32),
            pltpu.VMEM((N_DEV, k_per, n), F8),
            pltpu.VMEM((m_per, n), jnp.float32),
            pltpu.SemaphoreType.DMA((N_DEV,)),
            pltpu.SemaphoreType.DMA((2,)),
            pltpu.SemaphoreType.DMA((2,)),
            pltpu.SemaphoreType.DMA((N_DEV,)),
            pltpu.SemaphoreType.DMA((N_DEV,)),
            pltpu.SemaphoreType.DMA((N_DEV,)),
        ],
        compiler_params=pltpu.CompilerParams(
            collective_id=0,
            vmem_limit_bytes=64 * 1024 * 1024,
        ),
    )(x, w_mat, scale_x, scale_w)


# device time: 46709 ns/iter; 1.0274x vs baseline; 1.0073x over previous
import jax
import jax.numpy as jnp
from jax import lax
from jax.experimental import pallas as pl
from jax.experimental.pallas import tpu as pltpu

N_DEV = 4
N_Q = 4
F8 = jnp.float8_e4m3fn

ABLATE_NO_COMM = False

COMPUTE_ORDER = (0, 1, 3, 2)


def kernel(x, w_mat, scale_x, scale_w):
    m_total, k_shard = x.shape
    k_total, n = w_mat.shape
    m_per = m_total // N_DEV
    k_per = k_total // N_DEV
    m_q = m_per // N_Q

    def body(x_hbm_ref, w_hbm_ref, sx_ref, sw_ref, out_hbm_ref,
             xst_ref, xs_ref, xg_ref, wst_ref, w8_ref, acc_ref,
             x_sems, w_sems, out_sems, send_sems, recv_sems):
        my = lax.axis_index("i")
        diag = (my + 2) % N_DEV
        rings = [(my + 1) % N_DEV, (my + 3) % N_DEV]

        fetch = [diag, my] + rings
        x_copies = [
            pltpu.make_async_copy(
                x_hbm_ref.at[pl.ds(p * m_per, m_per), :],
                xst_ref.at[t],
                x_sems.at[t],
            )
            for t, p in enumerate(fetch)
        ]
        x_copies[0].start()

        barrier_sem = pltpu.get_barrier_semaphore()
        for j in range(N_DEV):
            @pl.when(j != my)
            def _():
                pl.semaphore_signal(
                    barrier_sem, inc=1,
                    device_id=(j,), device_id_type=pl.DeviceIdType.MESH,
                )
        pl.semaphore_wait(barrier_sem, N_DEV - 1)

        x_copies[0].wait()
        for cp in x_copies[1:]:
            cp.start()

        blocks = [(my + d) % N_DEV for d in COMPUTE_ORDER]
        w_copies = [
            pltpu.make_async_copy(
                w_hbm_ref.at[pl.ds(o * k_per, k_per), :],
                wst_ref.at[t % 2],
                w_sems.at[t % 2],
            )
            for t, o in enumerate(blocks)
        ]
        w_copies[0].start()
        w_copies[1].start()

        sends = []
        for q in range(N_Q):
            qs = pl.ds(q * m_q, m_q)
            xs_ref[0, qs] = xst_ref[0, qs].astype(F8)
            rdma = pltpu.make_async_remote_copy(
                src_ref=xs_ref.at[0, qs],
                dst_ref=xg_ref.at[my, qs],
                send_sem=send_sems.at[q],
                recv_sem=recv_sems.at[my, q],
                device_id=(diag,),
                device_id_type=pl.DeviceIdType.MESH,
            )
            if not ABLATE_NO_COMM:
                rdma.start()
                sends.append(rdma)

        x_copies[1].wait()
        xg_ref[my] = xst_ref[1].astype(F8)

        for i, p in enumerate(rings):
            x_copies[2 + i].wait()
            xs_ref[1 + i] = xst_ref[2 + i].astype(F8)
            rdma = pltpu.make_async_remote_copy(
                src_ref=xs_ref.at[1 + i],
                dst_ref=xg_ref.at[my],
                send_sem=send_sems.at[N_Q + i],
                recv_sem=recv_sems.at[my, 0],
                device_id=(p,),
                device_id_type=pl.DeviceIdType.MESH,
            )
            if not ABLATE_NO_COMM:
                rdma.start()
                sends.append(rdma)

        scale = sx_ref[0] * sw_ref[0]

        def block_dot(o, t, row_lo, rows):
            src = my if ABLATE_NO_COMM else o
            return lax.dot_general(
                xg_ref[src, pl.ds(row_lo, rows)], w8_ref[t],
                dimension_numbers=(((1,), (0,)), ((), ())),
                preferred_element_type=jnp.float32,
            ) * scale

        for t, o in enumerate(blocks[:3]):
            w_copies[t].wait()
            w8_ref[t] = wst_ref[t % 2].astype(F8)
            if t + 2 < N_DEV:
                w_copies[t + 2].start()

            if t > 0 and not ABLATE_NO_COMM:
                recv = pltpu.make_async_remote_copy(
                    src_ref=xs_ref.at[0],
                    dst_ref=xg_ref.at[o],
                    send_sem=send_sems.at[0],
                    recv_sem=recv_sems.at[o, 0],
                    device_id=(o,),
                    device_id_type=pl.DeviceIdType.MESH,
                )
                recv.wait_recv()

            contrib = block_dot(o, t, 0, m_per)
            if t == 0:
                acc_ref[:, :] = contrib
            else:
                acc_ref[:, :] = acc_ref[:, :] + contrib

        o = blocks[3]
        w_copies[3].wait()
        w8_ref[3] = wst_ref[3 % 2].astype(F8)
        out_cps = []
        for q in range(N_Q):
            qs = pl.ds(q * m_q, m_q)
            if not ABLATE_NO_COMM:
                recv = pltpu.make_async_remote_copy(
                    src_ref=xs_ref.at[0, qs],
                    dst_ref=xg_ref.at[o, qs],
                    send_sem=send_sems.at[0],
                    recv_sem=recv_sems.at[o, q],
                    device_id=(o,),
                    device_id_type=pl.DeviceIdType.MESH,
                )
                recv.wait_recv()
            acc_ref[qs, :] = acc_ref[qs, :] + block_dot(o, 3, q * m_q, m_q)
            cp = pltpu.make_async_copy(
                acc_ref.at[qs], out_hbm_ref.at[qs], out_sems.at[q],
            )
            cp.start()
            out_cps.append(cp)

        for rdma in sends:
            rdma.wait_send()
        for cp in out_cps:
            cp.wait()

    return pl.pallas_call(
        body,
        out_shape=jax.ShapeDtypeStruct((m_per, n), jnp.float32),
        in_specs=[
            pl.BlockSpec(memory_space=pl.ANY),
            pl.BlockSpec(memory_space=pl.ANY),
            pl.BlockSpec(memory_space=pltpu.SMEM),
            pl.BlockSpec(memory_space=pltpu.SMEM),
        ],
        out_specs=pl.BlockSpec(memory_space=pl.ANY),
        scratch_shapes=[
            pltpu.VMEM((N_DEV, m_per, k_shard), jnp.float32),
            pltpu.VMEM((N_DEV - 1, m_per, k_shard), F8),
            pltpu.VMEM((N_DEV, m_per, k_per), F8),
            pltpu.VMEM((2, k_per, n), jnp.float32),
            pltpu.VMEM((N_DEV, k_per, n), F8),
            pltpu.VMEM((m_per, n), jnp.float32),
            pltpu.SemaphoreType.DMA((N_DEV,)),
            pltpu.SemaphoreType.DMA((2,)),
            pltpu.SemaphoreType.DMA((N_Q,)),
            pltpu.SemaphoreType.DMA((N_Q + 2,)),
            pltpu.SemaphoreType.DMA((N_DEV, N_Q)),
        ],
        compiler_params=pltpu.CompilerParams(
            collective_id=0,
            vmem_limit_bytes=64 * 1024 * 1024,
        ),
    )(x, w_mat, scale_x, scale_w)


# device time: 45877 ns/iter; 1.0461x vs baseline; 1.0181x over previous
import jax
import jax.numpy as jnp
from jax import lax
from jax.experimental import pallas as pl
from jax.experimental.pallas import tpu as pltpu

N_DEV = 4
N_Q = 4
F8 = jnp.float8_e4m3fn

ABLATE_NO_COMM = False

COMPUTE_ORDER = (0, 1, 3, 2)


def kernel(x, w_mat, scale_x, scale_w):
    m_total, k_shard = x.shape
    k_total, n = w_mat.shape
    m_per = m_total // N_DEV
    k_per = k_total // N_DEV
    m_q = m_per // N_Q

    def body(x_hbm_ref, w_hbm_ref, sx_ref, sw_ref, out_hbm_ref,
             xst_ref, xs_ref, xg_ref, wst_ref, w8_ref, acc_ref,
             x_sems, w_sems, out_sems, send_sems, recv_sems):
        my = lax.axis_index("i")
        diag = (my + 2) % N_DEV
        rings = [(my + 1) % N_DEV, (my + 3) % N_DEV]

        fetch = [diag, my] + rings
        diag_row0 = diag * m_per
        x_head = pltpu.make_async_copy(
            x_hbm_ref.at[pl.ds(diag_row0, m_q), :],
            xst_ref.at[0, pl.ds(0, m_q)],
            x_sems.at[N_DEV],
        )
        x_tail = pltpu.make_async_copy(
            x_hbm_ref.at[pl.ds(diag_row0 + m_q, m_per - m_q), :],
            xst_ref.at[0, pl.ds(m_q, m_per - m_q)],
            x_sems.at[0],
        )
        x_head.start()
        x_tail.start()
        x_copies = [None] + [
            pltpu.make_async_copy(
                x_hbm_ref.at[pl.ds(p * m_per, m_per), :],
                xst_ref.at[t],
                x_sems.at[t],
            )
            for t, p in list(enumerate(fetch))[1:]
        ]

        barrier_sem = pltpu.get_barrier_semaphore()
        for j in range(N_DEV):
            @pl.when(j != my)
            def _():
                pl.semaphore_signal(
                    barrier_sem, inc=1,
                    device_id=(j,), device_id_type=pl.DeviceIdType.MESH,
                )
        pl.semaphore_wait(barrier_sem, N_DEV - 1)

        sends = []

        def send_diag_quarter(q):
            qs = pl.ds(q * m_q, m_q)
            xs_ref[0, qs] = xst_ref[0, qs].astype(F8)
            rdma = pltpu.make_async_remote_copy(
                src_ref=xs_ref.at[0, qs],
                dst_ref=xg_ref.at[my, qs],
                send_sem=send_sems.at[q],
                recv_sem=recv_sems.at[my, q],
                device_id=(diag,),
                device_id_type=pl.DeviceIdType.MESH,
            )
            if not ABLATE_NO_COMM:
                rdma.start()
                sends.append(rdma)

        x_head.wait()
        send_diag_quarter(0)

        for cp in x_copies[1:]:
            cp.start()
        blocks = [(my + d) % N_DEV for d in COMPUTE_ORDER]
        w_copies = [
            pltpu.make_async_copy(
                w_hbm_ref.at[pl.ds(o * k_per, k_per), :],
                wst_ref.at[t % 2],
                w_sems.at[t % 2],
            )
            for t, o in enumerate(blocks)
        ]
        w_copies[0].start()
        w_copies[1].start()

        x_tail.wait()
        for q in range(1, N_Q):
            send_diag_quarter(q)

        x_copies[1].wait()
        xg_ref[my] = xst_ref[1].astype(F8)

        for i, p in enumerate(rings):
            x_copies[2 + i].wait()
            xs_ref[1 + i] = xst_ref[2 + i].astype(F8)
            rdma = pltpu.make_async_remote_copy(
                src_ref=xs_ref.at[1 + i],
                dst_ref=xg_ref.at[my],
                send_sem=send_sems.at[N_Q + i],
                recv_sem=recv_sems.at[my, 0],
                device_id=(p,),
                device_id_type=pl.DeviceIdType.MESH,
            )
            if not ABLATE_NO_COMM:
                rdma.start()
                sends.append(rdma)

        scale = sx_ref[0] * sw_ref[0]

        def block_dot(o, t, row_lo, rows):
            src = my if ABLATE_NO_COMM else o
            return lax.dot_general(
                xg_ref[src, pl.ds(row_lo, rows)], w8_ref[t],
                dimension_numbers=(((1,), (0,)), ((), ())),
                preferred_element_type=jnp.float32,
            ) * scale

        for t, o in enumerate(blocks[:3]):
            w_copies[t].wait()
            w8_ref[t] = wst_ref[t % 2].astype(F8)
            if t + 2 < N_DEV:
                w_copies[t + 2].start()

            if t > 0 and not ABLATE_NO_COMM:
                recv = pltpu.make_async_remote_copy(
                    src_ref=xs_ref.at[0],
                    dst_ref=xg_ref.at[o],
                    send_sem=send_sems.at[0],
                    recv_sem=recv_sems.at[o, 0],
                    device_id=(o,),
                    device_id_type=pl.DeviceIdType.MESH,
                )
                recv.wait_recv()

            contrib = block_dot(o, t, 0, m_per)
            if t == 0:
                acc_ref[:, :] = contrib
            else:
                acc_ref[:, :] = acc_ref[:, :] + contrib

        o = blocks[3]
        w_copies[3].wait()
        w8_ref[3] = wst_ref[3 % 2].astype(F8)
        out_cps = []
        for q in range(N_Q):
            qs = pl.ds(q * m_q, m_q)
            if not ABLATE_NO_COMM:
                recv = pltpu.make_async_remote_copy(
                    src_ref=xs_ref.at[0, qs],
                    dst_ref=xg_ref.at[o, qs],
                    send_sem=send_sems.at[0],
                    recv_sem=recv_sems.at[o, q],
                    device_id=(o,),
                    device_id_type=pl.DeviceIdType.MESH,
                )
                recv.wait_recv()
            acc_ref[qs, :] = acc_ref[qs, :] + block_dot(o, 3, q * m_q, m_q)
            cp = pltpu.make_async_copy(
                acc_ref.at[qs], out_hbm_ref.at[qs], out_sems.at[q],
            )
            cp.start()
            out_cps.append(cp)

        for rdma in sends:
            rdma.wait_send()
        for cp in out_cps:
            cp.wait()

    return pl.pallas_call(
        body,
        out_shape=jax.ShapeDtypeStruct((m_per, n), jnp.float32),
        in_specs=[
            pl.BlockSpec(memory_space=pl.ANY),
            pl.BlockSpec(memory_space=pl.ANY),
            pl.BlockSpec(memory_space=pltpu.SMEM),
            pl.BlockSpec(memory_space=pltpu.SMEM),
        ],
        out_specs=pl.BlockSpec(memory_space=pl.ANY),
        scratch_shapes=[
            pltpu.VMEM((N_DEV, m_per, k_shard), jnp.float32),
            pltpu.VMEM((N_DEV - 1, m_per, k_shard), F8),
            pltpu.VMEM((N_DEV, m_per, k_per), F8),
            pltpu.VMEM((2, k_per, n), jnp.float32),
            pltpu.VMEM((N_DEV, k_per, n), F8),
            pltpu.VMEM((m_per, n), jnp.float32),
            pltpu.SemaphoreType.DMA((N_DEV + 1,)),
            pltpu.SemaphoreType.DMA((2,)),
            pltpu.SemaphoreType.DMA((N_Q,)),
            pltpu.SemaphoreType.DMA((N_Q + 2,)),
            pltpu.SemaphoreType.DMA((N_DEV, N_Q)),
        ],
        compiler_params=pltpu.CompilerParams(
            collective_id=0,
            vmem_limit_bytes=64 * 1024 * 1024,
        ),
    )(x, w_mat, scale_x, scale_w)
